# baseline (speedup 1.0000x reference)
import functools

import jax
import jax.numpy as jnp
from jax import lax
from jax.experimental import pallas as pl
from jax.experimental.pallas import tpu as pltpu
from jax.experimental.pallas import tpu_sc as plsc

VOCAB = 1000000
D_MODEL = 64

_info = plsc.get_sparse_core_info()
_NC, _NS = _info.num_cores, _info.num_subcores
_NW = _NC * _NS

_B = 4096 * 200
_BPW = _B // _NW
_CH = 512
_STEPS = _BPW // _CH


def _embed_kernel(x_hbm, table_hbm, out_hbm, idx_v, rows_v, sem):
    wid = lax.axis_index("s") * _NC + lax.axis_index("c")
    base = wid * _BPW

    def step(i, carry):
        off = base + i * _CH
        pltpu.sync_copy(x_hbm.at[pl.ds(off, _CH)], idx_v)
        pltpu.async_copy(table_hbm.at[idx_v], rows_v, sem).wait()
        pltpu.sync_copy(rows_v, out_hbm.at[pl.ds(off, _CH)])
        return carry

    lax.fori_loop(0, _STEPS, step, 0)


@jax.jit
def kernel(x, table):
    xf = x.reshape(-1).astype(jnp.int32)
    mesh = plsc.VectorSubcoreMesh(core_axis_name="c", subcore_axis_name="s")
    out = pl.kernel(
        _embed_kernel,
        mesh=mesh,
        out_type=jax.ShapeDtypeStruct((_B, D_MODEL), jnp.float32),
        scratch_types=[
            pltpu.VMEM((_CH,), jnp.int32),
            pltpu.VMEM((_CH, D_MODEL), jnp.float32),
            pltpu.SemaphoreType.DMA,
        ],
        compiler_params=pltpu.CompilerParams(use_tc_tiling_on_sc=False),
    )(xf, table)
    return out.reshape(x.shape[0], x.shape[1], D_MODEL)


# speedup vs baseline: 1.0433x; 1.0433x over previous
import jax
import jax.numpy as jnp
from jax import lax
from jax.experimental import pallas as pl
from jax.experimental.pallas import tpu as pltpu
from jax.experimental.pallas import tpu_sc as plsc

VOCAB = 1000000
D_MODEL = 64

_info = plsc.get_sparse_core_info()
_NC, _NS = _info.num_cores, _info.num_subcores
_NW = _NC * _NS

_B = 4096 * 200
_BPW = _B // _NW
_CH = 320
_STEPS = _BPW // _CH
_NBUF = 4
_ROUNDS = _STEPS // _NBUF


def _embed_kernel(x_hbm, table_hbm, out_hbm, idx_v, rows_v, gsem, wsem):
    wid = lax.axis_index("s") * _NC + lax.axis_index("c")
    base = wid * _BPW

    pltpu.sync_copy(x_hbm.at[pl.ds(base, _BPW)], idx_v)

    def gather_start(i, b):
        idx_slice = idx_v.at[pl.ds(i * _CH, _CH)]
        pltpu.async_copy(table_hbm.at[idx_slice], rows_v.at[b], gsem.at[b])

    def gather_wait(i, b):
        idx_slice = idx_v.at[pl.ds(i * _CH, _CH)]
        pltpu.make_async_copy(table_hbm.at[idx_slice], rows_v.at[b],
                              gsem.at[b]).wait()

    def wb_start(i, b):
        pltpu.async_copy(rows_v.at[b], out_hbm.at[pl.ds(base + i * _CH, _CH)],
                         wsem.at[b])

    def wb_wait(i, b):
        pltpu.make_async_copy(rows_v.at[b],
                              out_hbm.at[pl.ds(base + i * _CH, _CH)],
                              wsem.at[b]).wait()

    for b in range(_NBUF):
        gather_start(b, b)

    def round_body(r, carry):
        for b in range(_NBUF):
            i = r * _NBUF + b
            gather_wait(i, b)
            wb_start(i, b)
            wb_wait(i, b)
            gather_start(i + _NBUF, b)
        return carry

    lax.fori_loop(0, _ROUNDS - 1, round_body, 0)

    for b in range(_NBUF):
        i = (_ROUNDS - 1) * _NBUF + b
        gather_wait(i, b)
        wb_start(i, b)
        wb_wait(i, b)


@jax.jit
def kernel(x, table):
    xf = x.reshape(-1).astype(jnp.int32)
    mesh = plsc.VectorSubcoreMesh(core_axis_name="c", subcore_axis_name="s")
    out = pl.kernel(
        _embed_kernel,
        mesh=mesh,
        out_type=jax.ShapeDtypeStruct((_B, D_MODEL), jnp.float32),
        scratch_types=[
            pltpu.VMEM((_BPW,), jnp.int32),
            pltpu.VMEM((_NBUF, _CH, D_MODEL), jnp.float32),
            pltpu.SemaphoreType.DMA((_NBUF,)),
            pltpu.SemaphoreType.DMA((_NBUF,)),
        ],
        compiler_params=pltpu.CompilerParams(use_tc_tiling_on_sc=False),
    )(xf, table)
    return out.reshape(x.shape[0], x.shape[1], D_MODEL)


# speedup vs baseline: 1.0434x; 1.0001x over previous
import jax
import jax.numpy as jnp
from jax import lax
from jax.experimental import pallas as pl
from jax.experimental.pallas import tpu as pltpu
from jax.experimental.pallas import tpu_sc as plsc

VOCAB = 1000000
D_MODEL = 64

_info = plsc.get_sparse_core_info()
_NC, _NS = _info.num_cores, _info.num_subcores
_NW = _NC * _NS

_S = 4096
_T = 200
_SPW = _S // _NW
_NBUF = 4
_ROUNDS = _SPW // _NBUF


def _embed_kernel(x_hbm, table_hbm, out_hbm, idx_v, rows_v, gsem, wsem):
    wid = lax.axis_index("s") * _NC + lax.axis_index("c")
    s_base = wid * _SPW

    pltpu.sync_copy(x_hbm.at[pl.ds(s_base, _SPW)], idx_v)

    def gather_start(c, b):
        pltpu.async_copy(table_hbm.at[idx_v.at[c]], rows_v.at[b, 0],
                         gsem.at[b])

    def gather_wait(c, b):
        pltpu.make_async_copy(table_hbm.at[idx_v.at[c]], rows_v.at[b, 0],
                              gsem.at[b]).wait()

    def wb_start(c, b):
        pltpu.async_copy(rows_v.at[b], out_hbm.at[pl.ds(s_base + c, 1)],
                         wsem.at[b])

    def wb_wait(c, b):
        pltpu.make_async_copy(rows_v.at[b], out_hbm.at[pl.ds(s_base + c, 1)],
                              wsem.at[b]).wait()

    for b in range(_NBUF):
        gather_start(b, b)

    def round_body(r, carry):
        for b in range(_NBUF):
            c = r * _NBUF + b
            gather_wait(c, b)
            wb_start(c, b)
            wb_wait(c, b)
            gather_start(c + _NBUF, b)
        return carry

    lax.fori_loop(0, _ROUNDS - 1, round_body, 0)

    for b in range(_NBUF):
        c = (_ROUNDS - 1) * _NBUF + b
        gather_wait(c, b)
        wb_start(c, b)
        wb_wait(c, b)


@jax.jit
def kernel(x, table):
    mesh = plsc.VectorSubcoreMesh(core_axis_name="c", subcore_axis_name="s")
    out = pl.kernel(
        _embed_kernel,
        mesh=mesh,
        out_type=jax.ShapeDtypeStruct((_S, _T, D_MODEL), jnp.float32),
        scratch_types=[
            pltpu.VMEM((_SPW, _T), jnp.int32),
            pltpu.VMEM((_NBUF, 1, _T, D_MODEL), jnp.float32),
            pltpu.SemaphoreType.DMA((_NBUF,)),
            pltpu.SemaphoreType.DMA((_NBUF,)),
        ],
        compiler_params=pltpu.CompilerParams(use_tc_tiling_on_sc=False),
    )(x.astype(jnp.int32), table)
    return out
